# baseline (speedup 1.0000x reference)
import jax
import jax.numpy as jnp
from jax.experimental import pallas as pl


def _add_body(x_ref, pe_ref, o_ref):
    o_ref[...] = x_ref[...] + pe_ref[...][None, :, :]


def kernel(x, pe):
    B, L, D = x.shape
    bs = 1024
    grid = (L // bs,)
    return pl.pallas_call(
        _add_body,
        grid=grid,
        in_specs=[
            pl.BlockSpec((B, bs, D), lambda i: (0, i, 0)),
            pl.BlockSpec((bs, D), lambda i: (i, 0)),
        ],
        out_specs=pl.BlockSpec((B, bs, D), lambda i: (0, i, 0)),
        out_shape=jax.ShapeDtypeStruct((B, L, D), x.dtype),
    )(x, pe[:L])
